# baseline (speedup 1.0000x reference)
import functools

import jax
import jax.numpy as jnp
from jax import lax
from jax.experimental import pallas as pl
from jax.experimental.pallas import tpu as pltpu
from jax.experimental.pallas import tpu_sc as plsc

_B = 16384
_D = 32
_V = 1000000
_NC = 2
_NS = 16
_NW = _NC * _NS
_BPW = _B // _NW
_CH = _BPW // 16


def _gather_body(user_tab, item_tab, uidx, iidx, uout, iout,
                 uidx_v, iidx_v, uext_v, iext_v, urows_v, irows_v,
                 sem_u, sem_i):
    wid = lax.axis_index("s") * _NC + lax.axis_index("c")
    base = wid * _BPW
    pltpu.sync_copy(uidx.at[pl.ds(base, _BPW)], uidx_v)
    pltpu.sync_copy(iidx.at[pl.ds(base, _BPW)], iidx_v)

    def build(d, carry):
        off = d * _V
        for c in range(_CH):
            sl = pl.ds(c * 16, 16)
            uext_v[d, sl] = uidx_v[sl] + off
            iext_v[d, sl] = iidx_v[sl] + off
        return carry

    lax.fori_loop(0, _D, build, 0)

    ucopies = []
    icopies = []
    for d in range(_D):
        ucopies.append(
            pltpu.async_copy(user_tab.at[uext_v.at[d]], urows_v.at[d], sem_u))
        icopies.append(
            pltpu.async_copy(item_tab.at[iext_v.at[d]], irows_v.at[d], sem_i))
    for cp in ucopies:
        cp.wait()
    for cp in icopies:
        cp.wait()

    pltpu.sync_copy(urows_v, uout.at[:, pl.ds(base, _BPW)])
    pltpu.sync_copy(irows_v, iout.at[:, pl.ds(base, _BPW)])


@functools.lru_cache(maxsize=None)
def _build_gather2():
    mesh = plsc.VectorSubcoreMesh(
        core_axis_name="c", subcore_axis_name="s",
        num_cores=_NC, num_subcores=_NS,
    )
    return pl.kernel(
        _gather_body,
        out_type=(
            jax.ShapeDtypeStruct((_D, _B), jnp.float32),
            jax.ShapeDtypeStruct((_D, _B), jnp.float32),
        ),
        mesh=mesh,
        compiler_params=pltpu.CompilerParams(use_tc_tiling_on_sc=False),
        scratch_types=[
            pltpu.VMEM((_BPW,), jnp.int32),
            pltpu.VMEM((_BPW,), jnp.int32),
            pltpu.VMEM((_D, _BPW), jnp.int32),
            pltpu.VMEM((_D, _BPW), jnp.int32),
            pltpu.VMEM((_D, _BPW), jnp.float32),
            pltpu.VMEM((_D, _BPW), jnp.float32),
            pltpu.SemaphoreType.DMA,
            pltpu.SemaphoreType.DMA,
        ],
    )


_BS = 2048
_NB = _B // _BS


def _mlp_body(descT_ref, uT_ref, iT_ref,
              wdT_ref, bdT_ref, w1uT_ref, w1iT_ref, w1dT_ref, b1T_ref,
              w2T_ref, b2T_ref, woT_ref, bo_ref, out_ref):
    f32 = jnp.float32
    dT = lax.dot_general(wdT_ref[...], descT_ref[...], (((1,), (0,)), ((), ())),
                         preferred_element_type=f32)
    dT = jnp.maximum(dT + bdT_ref[...], 0.0)
    h1T = lax.dot_general(w1uT_ref[...], uT_ref[...], (((1,), (0,)), ((), ())),
                          preferred_element_type=f32)
    h1T = h1T + lax.dot_general(w1iT_ref[...], iT_ref[...], (((1,), (0,)), ((), ())),
                                preferred_element_type=f32)
    h1T = h1T + lax.dot_general(w1dT_ref[...], dT, (((1,), (0,)), ((), ())),
                                preferred_element_type=f32)
    h1T = jnp.maximum(h1T + b1T_ref[...], 0.0)
    h2T = lax.dot_general(w2T_ref[...], h1T, (((1,), (0,)), ((), ())),
                          preferred_element_type=f32)
    h2T = jnp.maximum(h2T + b2T_ref[...], 0.0)
    outT = lax.dot_general(woT_ref[...], h2T, (((1,), (0,)), ((), ())),
                           preferred_element_type=f32)
    out_ref[...] = (outT + bo_ref[...]).reshape(1, 1, _BS)


def _mlp(descT, uT, iT, wdT, bdT, w1uT, w1iT, w1dT, b1T, w2T, b2T, woT, bo):
    full = lambda shape: pl.BlockSpec(shape, lambda i: tuple(0 for _ in shape))
    return pl.pallas_call(
        _mlp_body,
        grid=(_NB,),
        in_specs=[
            pl.BlockSpec((300, _BS), lambda i: (0, i)),
            pl.BlockSpec((_D, _BS), lambda i: (0, i)),
            pl.BlockSpec((_D, _BS), lambda i: (0, i)),
            full((_D, 300)),
            full((_D, 1)),
            full((64, _D)),
            full((64, _D)),
            full((64, _D)),
            full((64, 1)),
            full((_D, 64)),
            full((_D, 1)),
            full((1, _D)),
            full((1, 1)),
        ],
        out_specs=pl.BlockSpec((1, 1, _BS), lambda i: (i, 0, 0)),
        out_shape=jax.ShapeDtypeStruct((_NB, 1, _BS), jnp.float32),
    )(descT, uT, iT, wdT, bdT, w1uT, w1iT, w1dT, b1T, w2T, b2T, woT, bo)


def kernel(user_input, item_input, description_input, user_table, item_table,
           W_desc, b_desc, W1, b1, W2, b2, W_out, b_out):
    uidx = user_input.reshape(-1)
    iidx = item_input.reshape(-1)
    utab_flat = user_table.T.reshape(-1)
    itab_flat = item_table.T.reshape(-1)
    uT, iT = _build_gather2()(utab_flat, itab_flat, uidx, iidx)
    W1T = W1.T
    out3 = _mlp(
        description_input.T, uT, iT,
        W_desc.T, b_desc.reshape(-1, 1),
        W1T[:, :_D], W1T[:, _D:2 * _D], W1T[:, 2 * _D:], b1.reshape(-1, 1),
        W2.T, b2.reshape(-1, 1),
        W_out.T, b_out.reshape(1, 1),
    )
    return out3.reshape(_B, 1)


# speedup vs baseline: 8.1993x; 8.1993x over previous
import functools

import jax
import jax.numpy as jnp
from jax import lax
from jax.experimental import pallas as pl
from jax.experimental.pallas import tpu as pltpu
from jax.experimental.pallas import tpu_sc as plsc

_B = 16384
_D = 32
_V = 1000000
_PACK = 4
_PD = _D * _PACK
_NC = 2
_NS = 16
_NW = _NC * _NS
_BPW = _B // _NW

_PR = 262144
_CK = 1024
_CG = _PR // _CK


def _compact_body(u0, u1, u2, u3, i0, i1, i2, i3, uout_ref, iout_ref):
    f32 = jnp.float32
    eye = (lax.broadcasted_iota(jnp.int32, (_D, _D), 0) ==
           lax.broadcasted_iota(jnp.int32, (_D, _D), 1)).astype(f32)

    def tr(ref):
        return lax.dot_general(ref[...], eye, (((0,), (0,)), ((), ())),
                               preferred_element_type=f32)

    uout_ref[...] = jnp.concatenate([tr(u0), tr(u1), tr(u2), tr(u3)], axis=1)
    iout_ref[...] = jnp.concatenate([tr(i0), tr(i1), tr(i2), tr(i3)], axis=1)


def _compact(utabT, itabT):
    in_specs = []
    last_blk = (_V - 1) // _CK
    for _ in range(2):
        for k in range(_PACK):
            in_specs.append(
                pl.BlockSpec(
                    (_D, _CK),
                    functools.partial(
                        lambda i, kk: (0, jnp.minimum(i + kk * _CG, last_blk)),
                        kk=k)))
    return pl.pallas_call(
        _compact_body,
        grid=(_CG,),
        in_specs=in_specs,
        out_specs=[
            pl.BlockSpec((_CK, _PD), lambda i: (i, 0)),
            pl.BlockSpec((_CK, _PD), lambda i: (i, 0)),
        ],
        out_shape=[
            jax.ShapeDtypeStruct((_PR, _PD), jnp.float32),
            jax.ShapeDtypeStruct((_PR, _PD), jnp.float32),
        ],
    )(utabT, utabT, utabT, utabT, itabT, itabT, itabT, itabT)


def _gather_body(user_tab, item_tab, uidx, iidx, uout, iout,
                 uidx_v, iidx_v, rows_v, sem):
    wid = lax.axis_index("s") * _NC + lax.axis_index("c")
    base = wid * _BPW
    pltpu.sync_copy(uidx.at[pl.ds(base, _BPW)], uidx_v)
    pltpu.sync_copy(iidx.at[pl.ds(base, _BPW)], iidx_v)
    pltpu.async_copy(user_tab.at[uidx_v], rows_v, sem).wait()
    pltpu.sync_copy(rows_v, uout.at[pl.ds(base, _BPW)])
    pltpu.async_copy(item_tab.at[iidx_v], rows_v, sem).wait()
    pltpu.sync_copy(rows_v, iout.at[pl.ds(base, _BPW)])


@functools.lru_cache(maxsize=None)
def _build_gather2():
    mesh = plsc.VectorSubcoreMesh(
        core_axis_name="c", subcore_axis_name="s",
        num_cores=_NC, num_subcores=_NS,
    )
    return pl.kernel(
        _gather_body,
        out_type=(
            jax.ShapeDtypeStruct((_B, _PD), jnp.float32),
            jax.ShapeDtypeStruct((_B, _PD), jnp.float32),
        ),
        mesh=mesh,
        scratch_types=[
            pltpu.VMEM((_BPW,), jnp.int32),
            pltpu.VMEM((_BPW,), jnp.int32),
            pltpu.VMEM((_BPW, _PD), jnp.float32),
            pltpu.SemaphoreType.DMA,
        ],
    )


_BS = 2048
_NB = _B // _BS


def _mlp_body(descT_ref, uraw_ref, iraw_ref, uoff_ref, ioff_ref,
              wdT_ref, bdT_ref, w1uT_ref, w1iT_ref, w1dT_ref, b1T_ref,
              w2T_ref, b2T_ref, woT_ref, bo_ref, out_ref):
    f32 = jnp.float32
    uraw = uraw_ref[...]
    iraw = iraw_ref[...]
    uoff = uoff_ref[...]
    ioff = ioff_ref[...]
    u = jnp.zeros((_BS, _D), f32)
    it = jnp.zeros((_BS, _D), f32)
    for k in range(_PACK):
        umask = (uoff == k).astype(f32)
        imask = (ioff == k).astype(f32)
        u = u + umask * uraw[:, k * _D:(k + 1) * _D]
        it = it + imask * iraw[:, k * _D:(k + 1) * _D]
    dT = lax.dot_general(wdT_ref[...], descT_ref[...], (((1,), (0,)), ((), ())),
                         preferred_element_type=f32)
    dT = jnp.maximum(dT + bdT_ref[...], 0.0)
    h1T = lax.dot_general(w1uT_ref[...], u, (((1,), (1,)), ((), ())),
                          preferred_element_type=f32)
    h1T = h1T + lax.dot_general(w1iT_ref[...], it, (((1,), (1,)), ((), ())),
                                preferred_element_type=f32)
    h1T = h1T + lax.dot_general(w1dT_ref[...], dT, (((1,), (0,)), ((), ())),
                                preferred_element_type=f32)
    h1T = jnp.maximum(h1T + b1T_ref[...], 0.0)
    h2T = lax.dot_general(w2T_ref[...], h1T, (((1,), (0,)), ((), ())),
                          preferred_element_type=f32)
    h2T = jnp.maximum(h2T + b2T_ref[...], 0.0)
    outT = lax.dot_general(woT_ref[...], h2T, (((1,), (0,)), ((), ())),
                           preferred_element_type=f32)
    out_ref[...] = (outT + bo_ref[...]).reshape(1, 1, _BS)


def _mlp(descT, u_raw, i_raw, uoff, ioff, wdT, bdT, w1uT, w1iT, w1dT, b1T,
         w2T, b2T, woT, bo):
    full = lambda shape: pl.BlockSpec(shape, lambda i: tuple(0 for _ in shape))
    return pl.pallas_call(
        _mlp_body,
        grid=(_NB,),
        in_specs=[
            pl.BlockSpec((300, _BS), lambda i: (0, i)),
            pl.BlockSpec((_BS, _PD), lambda i: (i, 0)),
            pl.BlockSpec((_BS, _PD), lambda i: (i, 0)),
            pl.BlockSpec((_BS, 1), lambda i: (i, 0)),
            pl.BlockSpec((_BS, 1), lambda i: (i, 0)),
            full((_D, 300)),
            full((_D, 1)),
            full((64, _D)),
            full((64, _D)),
            full((64, _D)),
            full((64, 1)),
            full((_D, 64)),
            full((_D, 1)),
            full((1, _D)),
            full((1, 1)),
        ],
        out_specs=pl.BlockSpec((1, 1, _BS), lambda i: (i, 0, 0)),
        out_shape=jax.ShapeDtypeStruct((_NB, 1, _BS), jnp.float32),
    )(descT, u_raw, i_raw, uoff, ioff, wdT, bdT, w1uT, w1iT, w1dT, b1T,
      w2T, b2T, woT, bo)


def kernel(user_input, item_input, description_input, user_table, item_table,
           W_desc, b_desc, W1, b1, W2, b2, W_out, b_out):
    utab4, itab4 = _compact(user_table.T, item_table.T)
    uidx = user_input.reshape(-1)
    iidx = item_input.reshape(-1)
    u_raw, i_raw = _build_gather2()(utab4, itab4,
                                    jnp.bitwise_and(uidx, _PR - 1),
                                    jnp.bitwise_and(iidx, _PR - 1))
    uoff = lax.shift_right_logical(user_input, 18).astype(jnp.int32)
    ioff = lax.shift_right_logical(item_input, 18).astype(jnp.int32)
    W1T = W1.T
    out3 = _mlp(
        description_input.T, u_raw, i_raw, uoff, ioff,
        W_desc.T, b_desc.reshape(-1, 1),
        W1T[:, :_D], W1T[:, _D:2 * _D], W1T[:, 2 * _D:], b1.reshape(-1, 1),
        W2.T, b2.reshape(-1, 1),
        W_out.T, b_out.reshape(1, 1),
    )
    return out3.reshape(_B, 1)


# speedup vs baseline: 12.7588x; 1.5561x over previous
import functools

import jax
import jax.numpy as jnp
from jax import lax
from jax.experimental import pallas as pl
from jax.experimental.pallas import tpu as pltpu
from jax.experimental.pallas import tpu_sc as plsc

_B = 16384
_D = 32
_V = 1000000
_PACK = 4
_PD = _D * _PACK
_NC = 2
_NS = 16
_NW = _NC * _NS
_BPW = _B // _NW

_PR = 262144
_CK = 2048
_CG = _PR // _CK


def _compact_body(u0, u1, u2, u3, i0, i1, i2, i3, uout_ref, iout_ref):
    f32 = jnp.float32
    col = lax.broadcasted_iota(jnp.int32, (_D, _PD), 1)
    row = lax.broadcasted_iota(jnp.int32, (_D, _PD), 0)

    def place(refs):
        acc = None
        for k, ref in enumerate(refs):
            ek = (col == row + 32 * k).astype(f32)
            y = lax.dot_general(ref[...], ek, (((0,), (0,)), ((), ())),
                                preferred_element_type=f32)
            acc = y if acc is None else acc + y
        return acc

    uout_ref[...] = place([u0, u1, u2, u3])
    iout_ref[...] = place([i0, i1, i2, i3])


def _compact(utabT, itabT):
    in_specs = []
    last_blk = (_V - 1) // _CK
    for _ in range(2):
        for k in range(_PACK):
            in_specs.append(
                pl.BlockSpec(
                    (_D, _CK),
                    functools.partial(
                        lambda i, kk: (0, jnp.minimum(i + kk * _CG, last_blk)),
                        kk=k)))
    return pl.pallas_call(
        _compact_body,
        grid=(_CG,),
        in_specs=in_specs,
        out_specs=[
            pl.BlockSpec((_CK, _PD), lambda i: (i, 0)),
            pl.BlockSpec((_CK, _PD), lambda i: (i, 0)),
        ],
        out_shape=[
            jax.ShapeDtypeStruct((_PR, _PD), jnp.float32),
            jax.ShapeDtypeStruct((_PR, _PD), jnp.float32),
        ],
        compiler_params=pltpu.CompilerParams(fuse_transposed_lhs_in_matmul=True),
    )(utabT, utabT, utabT, utabT, itabT, itabT, itabT, itabT)


def _gather_body(user_tab, item_tab, uidx, iidx, uout, iout,
                 uidx_v, iidx_v, rows_v, sem):
    wid = lax.axis_index("s") * _NC + lax.axis_index("c")
    base = wid * _BPW
    pltpu.sync_copy(uidx.at[pl.ds(base, _BPW)], uidx_v)
    pltpu.sync_copy(iidx.at[pl.ds(base, _BPW)], iidx_v)
    pltpu.async_copy(user_tab.at[uidx_v], rows_v, sem).wait()
    pltpu.sync_copy(rows_v, uout.at[pl.ds(base, _BPW)])
    pltpu.async_copy(item_tab.at[iidx_v], rows_v, sem).wait()
    pltpu.sync_copy(rows_v, iout.at[pl.ds(base, _BPW)])


@functools.lru_cache(maxsize=None)
def _build_gather2():
    mesh = plsc.VectorSubcoreMesh(
        core_axis_name="c", subcore_axis_name="s",
        num_cores=_NC, num_subcores=_NS,
    )
    return pl.kernel(
        _gather_body,
        out_type=(
            jax.ShapeDtypeStruct((_B, _PD), jnp.float32),
            jax.ShapeDtypeStruct((_B, _PD), jnp.float32),
        ),
        mesh=mesh,
        scratch_types=[
            pltpu.VMEM((_BPW,), jnp.int32),
            pltpu.VMEM((_BPW,), jnp.int32),
            pltpu.VMEM((_BPW, _PD), jnp.float32),
            pltpu.SemaphoreType.DMA,
        ],
    )


_BS = 2048
_NB = _B // _BS


def _mlp_body(descT_ref, uraw_ref, iraw_ref, uoff_ref, ioff_ref,
              wdT_ref, bdT_ref, w1uT_ref, w1iT_ref, w1dT_ref, b1T_ref,
              w2T_ref, b2T_ref, woT_ref, bo_ref, out_ref):
    f32 = jnp.float32
    uraw = uraw_ref[...]
    iraw = iraw_ref[...]
    uoff = uoff_ref[...]
    ioff = ioff_ref[...]
    u = jnp.zeros((_BS, _D), f32)
    it = jnp.zeros((_BS, _D), f32)
    for k in range(_PACK):
        umask = (uoff == k).astype(f32)
        imask = (ioff == k).astype(f32)
        u = u + umask * uraw[:, k * _D:(k + 1) * _D]
        it = it + imask * iraw[:, k * _D:(k + 1) * _D]
    dT = lax.dot_general(wdT_ref[...], descT_ref[...], (((1,), (0,)), ((), ())),
                         preferred_element_type=f32)
    dT = jnp.maximum(dT + bdT_ref[...], 0.0)
    h1T = lax.dot_general(w1uT_ref[...], u, (((1,), (1,)), ((), ())),
                          preferred_element_type=f32)
    h1T = h1T + lax.dot_general(w1iT_ref[...], it, (((1,), (1,)), ((), ())),
                                preferred_element_type=f32)
    h1T = h1T + lax.dot_general(w1dT_ref[...], dT, (((1,), (0,)), ((), ())),
                                preferred_element_type=f32)
    h1T = jnp.maximum(h1T + b1T_ref[...], 0.0)
    h2T = lax.dot_general(w2T_ref[...], h1T, (((1,), (0,)), ((), ())),
                          preferred_element_type=f32)
    h2T = jnp.maximum(h2T + b2T_ref[...], 0.0)
    outT = lax.dot_general(woT_ref[...], h2T, (((1,), (0,)), ((), ())),
                           preferred_element_type=f32)
    out_ref[...] = (outT + bo_ref[...]).reshape(1, 1, _BS)


def _mlp(descT, u_raw, i_raw, uoff, ioff, wdT, bdT, w1uT, w1iT, w1dT, b1T,
         w2T, b2T, woT, bo):
    full = lambda shape: pl.BlockSpec(shape, lambda i: tuple(0 for _ in shape))
    return pl.pallas_call(
        _mlp_body,
        grid=(_NB,),
        in_specs=[
            pl.BlockSpec((300, _BS), lambda i: (0, i)),
            pl.BlockSpec((_BS, _PD), lambda i: (i, 0)),
            pl.BlockSpec((_BS, _PD), lambda i: (i, 0)),
            pl.BlockSpec((_BS, 1), lambda i: (i, 0)),
            pl.BlockSpec((_BS, 1), lambda i: (i, 0)),
            full((_D, 300)),
            full((_D, 1)),
            full((64, _D)),
            full((64, _D)),
            full((64, _D)),
            full((64, 1)),
            full((_D, 64)),
            full((_D, 1)),
            full((1, _D)),
            full((1, 1)),
        ],
        out_specs=pl.BlockSpec((1, 1, _BS), lambda i: (i, 0, 0)),
        out_shape=jax.ShapeDtypeStruct((_NB, 1, _BS), jnp.float32),
    )(descT, u_raw, i_raw, uoff, ioff, wdT, bdT, w1uT, w1iT, w1dT, b1T,
      w2T, b2T, woT, bo)


def kernel(user_input, item_input, description_input, user_table, item_table,
           W_desc, b_desc, W1, b1, W2, b2, W_out, b_out):
    utab4, itab4 = _compact(user_table.T, item_table.T)
    uidx = user_input.reshape(-1)
    iidx = item_input.reshape(-1)
    u_raw, i_raw = _build_gather2()(utab4, itab4,
                                    jnp.bitwise_and(uidx, _PR - 1),
                                    jnp.bitwise_and(iidx, _PR - 1))
    uoff = lax.shift_right_logical(user_input, 18).astype(jnp.int32)
    ioff = lax.shift_right_logical(item_input, 18).astype(jnp.int32)
    W1T = W1.T
    out3 = _mlp(
        description_input.T, u_raw, i_raw, uoff, ioff,
        W_desc.T, b_desc.reshape(-1, 1),
        W1T[:, :_D], W1T[:, _D:2 * _D], W1T[:, 2 * _D:], b1.reshape(-1, 1),
        W2.T, b2.reshape(-1, 1),
        W_out.T, b_out.reshape(1, 1),
    )
    return out3.reshape(_B, 1)


# speedup vs baseline: 18.1904x; 1.4257x over previous
import functools

import jax
import jax.numpy as jnp
from jax import lax
from jax.experimental import pallas as pl
from jax.experimental.pallas import tpu as pltpu
from jax.experimental.pallas import tpu_sc as plsc

_B = 16384
_D = 32
_V = 1000000
_PACK = 4
_PD = _D * _PACK
_NC = 2
_NS = 16
_NW = _NC * _NS
_BPW = _B // _NW

_PR = 262144
_CK = 2048
_CG = _PR // _CK


def _compact_body(u0, u1, u2, u3, i0, i1, i2, i3, uout_ref, iout_ref):
    f32 = jnp.float32
    eye = (lax.broadcasted_iota(jnp.int32, (_PD, _PD), 0) ==
           lax.broadcasted_iota(jnp.int32, (_PD, _PD), 1)).astype(f32)

    def place(refs, out_ref):
        x = jnp.concatenate([r[...] for r in refs], axis=0)
        out_ref[...] = lax.dot_general(x, eye, (((0,), (0,)), ((), ())),
                                       preferred_element_type=f32)

    place([u0, u1, u2, u3], uout_ref)
    place([i0, i1, i2, i3], iout_ref)


def _compact(utabT, itabT):
    in_specs = []
    last_blk = (_V - 1) // _CK
    for _ in range(2):
        for k in range(_PACK):
            in_specs.append(
                pl.BlockSpec(
                    (_D, _CK),
                    functools.partial(
                        lambda i, kk: (0, jnp.minimum(i + kk * _CG, last_blk)),
                        kk=k)))
    return pl.pallas_call(
        _compact_body,
        grid=(_CG,),
        in_specs=in_specs,
        out_specs=[
            pl.BlockSpec((_CK, _PD), lambda i: (i, 0)),
            pl.BlockSpec((_CK, _PD), lambda i: (i, 0)),
        ],
        out_shape=[
            jax.ShapeDtypeStruct((_PR, _PD), jnp.float32),
            jax.ShapeDtypeStruct((_PR, _PD), jnp.float32),
        ],
        compiler_params=pltpu.CompilerParams(fuse_transposed_lhs_in_matmul=True),
    )(utabT, utabT, utabT, utabT, itabT, itabT, itabT, itabT)


def _gather_body(user_tab, item_tab, uidx, iidx, uout, iout,
                 uidx_v, iidx_v, rows_v, sem):
    wid = lax.axis_index("s") * _NC + lax.axis_index("c")
    base = wid * _BPW
    pltpu.sync_copy(uidx.at[pl.ds(base, _BPW)], uidx_v)
    pltpu.sync_copy(iidx.at[pl.ds(base, _BPW)], iidx_v)
    pltpu.async_copy(user_tab.at[uidx_v], rows_v, sem).wait()
    pltpu.sync_copy(rows_v, uout.at[pl.ds(base, _BPW)])
    pltpu.async_copy(item_tab.at[iidx_v], rows_v, sem).wait()
    pltpu.sync_copy(rows_v, iout.at[pl.ds(base, _BPW)])


@functools.lru_cache(maxsize=None)
def _build_gather2():
    mesh = plsc.VectorSubcoreMesh(
        core_axis_name="c", subcore_axis_name="s",
        num_cores=_NC, num_subcores=_NS,
    )
    return pl.kernel(
        _gather_body,
        out_type=(
            jax.ShapeDtypeStruct((_B, _PD), jnp.float32),
            jax.ShapeDtypeStruct((_B, _PD), jnp.float32),
        ),
        mesh=mesh,
        scratch_types=[
            pltpu.VMEM((_BPW,), jnp.int32),
            pltpu.VMEM((_BPW,), jnp.int32),
            pltpu.VMEM((_BPW, _PD), jnp.float32),
            pltpu.SemaphoreType.DMA,
        ],
    )


_BS = 2048
_NB = _B // _BS


def _mlp_body(descT_ref, uraw_ref, iraw_ref, uoff_ref, ioff_ref,
              wdT_ref, bdT_ref, w1uT_ref, w1iT_ref, w1dT_ref, b1T_ref,
              w2T_ref, b2T_ref, woT_ref, bo_ref, out_ref):
    f32 = jnp.float32
    uraw = uraw_ref[...]
    iraw = iraw_ref[...]
    uoff = uoff_ref[...]
    ioff = ioff_ref[...]
    u = jnp.zeros((_BS, _D), f32)
    it = jnp.zeros((_BS, _D), f32)
    for k in range(_PACK):
        umask = (uoff == k).astype(f32)
        imask = (ioff == k).astype(f32)
        u = u + umask * uraw[:, k * _D:(k + 1) * _D]
        it = it + imask * iraw[:, k * _D:(k + 1) * _D]
    dT = lax.dot_general(wdT_ref[...], descT_ref[...], (((1,), (0,)), ((), ())),
                         preferred_element_type=f32)
    dT = jnp.maximum(dT + bdT_ref[...], 0.0)
    h1T = lax.dot_general(w1uT_ref[...], u, (((1,), (1,)), ((), ())),
                          preferred_element_type=f32)
    h1T = h1T + lax.dot_general(w1iT_ref[...], it, (((1,), (1,)), ((), ())),
                                preferred_element_type=f32)
    h1T = h1T + lax.dot_general(w1dT_ref[...], dT, (((1,), (0,)), ((), ())),
                                preferred_element_type=f32)
    h1T = jnp.maximum(h1T + b1T_ref[...], 0.0)
    h2T = lax.dot_general(w2T_ref[...], h1T, (((1,), (0,)), ((), ())),
                          preferred_element_type=f32)
    h2T = jnp.maximum(h2T + b2T_ref[...], 0.0)
    outT = lax.dot_general(woT_ref[...], h2T, (((1,), (0,)), ((), ())),
                           preferred_element_type=f32)
    out_ref[...] = (outT + bo_ref[...]).reshape(1, 1, _BS)


def _mlp(descT, u_raw, i_raw, uoff, ioff, wdT, bdT, w1uT, w1iT, w1dT, b1T,
         w2T, b2T, woT, bo):
    full = lambda shape: pl.BlockSpec(shape, lambda i: tuple(0 for _ in shape))
    return pl.pallas_call(
        _mlp_body,
        grid=(_NB,),
        in_specs=[
            pl.BlockSpec((300, _BS), lambda i: (0, i)),
            pl.BlockSpec((_BS, _PD), lambda i: (i, 0)),
            pl.BlockSpec((_BS, _PD), lambda i: (i, 0)),
            pl.BlockSpec((_BS, 1), lambda i: (i, 0)),
            pl.BlockSpec((_BS, 1), lambda i: (i, 0)),
            full((_D, 300)),
            full((_D, 1)),
            full((64, _D)),
            full((64, _D)),
            full((64, _D)),
            full((64, 1)),
            full((_D, 64)),
            full((_D, 1)),
            full((1, _D)),
            full((1, 1)),
        ],
        out_specs=pl.BlockSpec((1, 1, _BS), lambda i: (i, 0, 0)),
        out_shape=jax.ShapeDtypeStruct((_NB, 1, _BS), jnp.float32),
    )(descT, u_raw, i_raw, uoff, ioff, wdT, bdT, w1uT, w1iT, w1dT, b1T,
      w2T, b2T, woT, bo)


def kernel(user_input, item_input, description_input, user_table, item_table,
           W_desc, b_desc, W1, b1, W2, b2, W_out, b_out):
    utab4, itab4 = _compact(user_table.T, item_table.T)
    uidx = user_input.reshape(-1)
    iidx = item_input.reshape(-1)
    u_raw, i_raw = _build_gather2()(utab4, itab4,
                                    jnp.bitwise_and(uidx, _PR - 1),
                                    jnp.bitwise_and(iidx, _PR - 1))
    uoff = lax.shift_right_logical(user_input, 18).astype(jnp.int32)
    ioff = lax.shift_right_logical(item_input, 18).astype(jnp.int32)
    W1T = W1.T
    out3 = _mlp(
        description_input.T, u_raw, i_raw, uoff, ioff,
        W_desc.T, b_desc.reshape(-1, 1),
        W1T[:, :_D], W1T[:, _D:2 * _D], W1T[:, 2 * _D:], b1.reshape(-1, 1),
        W2.T, b2.reshape(-1, 1),
        W_out.T, b_out.reshape(1, 1),
    )
    return out3.reshape(_B, 1)


# speedup vs baseline: 19.2698x; 1.0593x over previous
import functools

import jax
import jax.numpy as jnp
from jax import lax
from jax.experimental import pallas as pl
from jax.experimental.pallas import tpu as pltpu
from jax.experimental.pallas import tpu_sc as plsc

_B = 16384
_D = 32
_V = 1000000
_PACK = 4
_PD = _D * _PACK
_NC = 2
_NS = 16
_NW = _NC * _NS
_BPW = _B // _NW

_PR = 262144
_CK = 2048
_CG = _PR // _CK


def _compact_body(u0, u1, u2, u3, i0, i1, i2, i3, uout_ref, iout_ref):
    f32 = jnp.float32
    eye = (lax.broadcasted_iota(jnp.int32, (_PD, _PD), 0) ==
           lax.broadcasted_iota(jnp.int32, (_PD, _PD), 1)).astype(f32)

    def place(refs, out_ref):
        x = jnp.concatenate([r[...] for r in refs], axis=0)
        out_ref[...] = lax.dot_general(x, eye, (((0,), (0,)), ((), ())),
                                       preferred_element_type=f32)

    place([u0, u1, u2, u3], uout_ref)
    place([i0, i1, i2, i3], iout_ref)


def _compact(utabT, itabT):
    in_specs = []
    last_blk = (_V - 1) // _CK
    for _ in range(2):
        for k in range(_PACK):
            in_specs.append(
                pl.BlockSpec(
                    (_D, _CK),
                    functools.partial(
                        lambda i, kk: (0, jnp.minimum(i + kk * _CG, last_blk)),
                        kk=k)))
    return pl.pallas_call(
        _compact_body,
        grid=(_CG,),
        in_specs=in_specs,
        out_specs=[
            pl.BlockSpec((_CK, _PD), lambda i: (i, 0)),
            pl.BlockSpec((_CK, _PD), lambda i: (i, 0)),
        ],
        out_shape=[
            jax.ShapeDtypeStruct((_PR, _PD), jnp.float32),
            jax.ShapeDtypeStruct((_PR, _PD), jnp.float32),
        ],
        compiler_params=pltpu.CompilerParams(fuse_transposed_lhs_in_matmul=True),
    )(utabT, utabT, utabT, utabT, itabT, itabT, itabT, itabT)


def _gather_body(user_tab, item_tab, uidx, iidx, uout, iout,
                 uidx_v, iidx_v, rows_v, sem):
    wid = lax.axis_index("s") * _NC + lax.axis_index("c")
    base = wid * _BPW
    pltpu.sync_copy(uidx.at[pl.ds(base, _BPW)], uidx_v)
    pltpu.sync_copy(iidx.at[pl.ds(base, _BPW)], iidx_v)
    for c in range(_BPW // 16):
        sl = pl.ds(c * 16, 16)
        uidx_v[sl] = jnp.bitwise_and(uidx_v[sl], _PR - 1)
        iidx_v[sl] = jnp.bitwise_and(iidx_v[sl], _PR - 1)
    pltpu.async_copy(user_tab.at[uidx_v], rows_v, sem).wait()
    pltpu.sync_copy(rows_v, uout.at[pl.ds(base, _BPW)])
    pltpu.async_copy(item_tab.at[iidx_v], rows_v, sem).wait()
    pltpu.sync_copy(rows_v, iout.at[pl.ds(base, _BPW)])


@functools.lru_cache(maxsize=None)
def _build_gather2():
    mesh = plsc.VectorSubcoreMesh(
        core_axis_name="c", subcore_axis_name="s",
        num_cores=_NC, num_subcores=_NS,
    )
    return pl.kernel(
        _gather_body,
        out_type=(
            jax.ShapeDtypeStruct((_B, _PD), jnp.float32),
            jax.ShapeDtypeStruct((_B, _PD), jnp.float32),
        ),
        mesh=mesh,
        scratch_types=[
            pltpu.VMEM((_BPW,), jnp.int32),
            pltpu.VMEM((_BPW,), jnp.int32),
            pltpu.VMEM((_BPW, _PD), jnp.float32),
            pltpu.SemaphoreType.DMA,
        ],
    )


_BS = 2048
_NB = _B // _BS


def _mlp_body(descT_ref, uraw_ref, iraw_ref, uoff_ref, ioff_ref,
              wdT_ref, bdT_ref, w1uT_ref, w1iT_ref, w1dT_ref, b1T_ref,
              w2T_ref, b2T_ref, woT_ref, bo_ref, out_ref):
    f32 = jnp.float32
    uraw = uraw_ref[...]
    iraw = iraw_ref[...]
    uoff = lax.shift_right_logical(jnp.transpose(uoff_ref[...]), 18)
    ioff = lax.shift_right_logical(jnp.transpose(ioff_ref[...]), 18)
    u = jnp.zeros((_BS, _D), f32)
    it = jnp.zeros((_BS, _D), f32)
    for k in range(_PACK):
        umask = (uoff == k).astype(f32)
        imask = (ioff == k).astype(f32)
        u = u + umask * uraw[:, k * _D:(k + 1) * _D]
        it = it + imask * iraw[:, k * _D:(k + 1) * _D]
    dT = lax.dot_general(wdT_ref[...], descT_ref[...], (((1,), (0,)), ((), ())),
                         preferred_element_type=f32)
    dT = jnp.maximum(dT + bdT_ref[...], 0.0)
    h1T = lax.dot_general(w1uT_ref[...], u, (((1,), (1,)), ((), ())),
                          preferred_element_type=f32)
    h1T = h1T + lax.dot_general(w1iT_ref[...], it, (((1,), (1,)), ((), ())),
                                preferred_element_type=f32)
    h1T = h1T + lax.dot_general(w1dT_ref[...], dT, (((1,), (0,)), ((), ())),
                                preferred_element_type=f32)
    h1T = jnp.maximum(h1T + b1T_ref[...], 0.0)
    h2T = lax.dot_general(w2T_ref[...], h1T, (((1,), (0,)), ((), ())),
                          preferred_element_type=f32)
    h2T = jnp.maximum(h2T + b2T_ref[...], 0.0)
    outT = lax.dot_general(woT_ref[...], h2T, (((1,), (0,)), ((), ())),
                           preferred_element_type=f32)
    out_ref[...] = (outT + bo_ref[...]).reshape(1, 1, _BS)


def _mlp(descT, u_raw, i_raw, uoff, ioff, wdT, bdT, w1uT, w1iT, w1dT, b1T,
         w2T, b2T, woT, bo):
    full = lambda shape: pl.BlockSpec(shape, lambda i: tuple(0 for _ in shape))
    return pl.pallas_call(
        _mlp_body,
        grid=(_NB,),
        in_specs=[
            pl.BlockSpec((300, _BS), lambda i: (0, i)),
            pl.BlockSpec((_BS, _PD), lambda i: (i, 0)),
            pl.BlockSpec((_BS, _PD), lambda i: (i, 0)),
            pl.BlockSpec((1, _BS), lambda i: (0, i)),
            pl.BlockSpec((1, _BS), lambda i: (0, i)),
            full((_D, 300)),
            full((_D, 1)),
            full((64, _D)),
            full((64, _D)),
            full((64, _D)),
            full((64, 1)),
            full((_D, 64)),
            full((_D, 1)),
            full((1, _D)),
            full((1, 1)),
        ],
        out_specs=pl.BlockSpec((1, 1, _BS), lambda i: (i, 0, 0)),
        out_shape=jax.ShapeDtypeStruct((_NB, 1, _BS), jnp.float32),
    )(descT, u_raw, i_raw, uoff, ioff, wdT, bdT, w1uT, w1iT, w1dT, b1T,
      w2T, b2T, woT, bo)


def kernel(user_input, item_input, description_input, user_table, item_table,
           W_desc, b_desc, W1, b1, W2, b2, W_out, b_out):
    utab4, itab4 = _compact(user_table.T, item_table.T)
    uidx = user_input.reshape(-1)
    iidx = item_input.reshape(-1)
    u_raw, i_raw = _build_gather2()(utab4, itab4, uidx, iidx)
    uoff = user_input.T
    ioff = item_input.T
    W1T = W1.T
    out3 = _mlp(
        description_input.T, u_raw, i_raw, uoff, ioff,
        W_desc.T, b_desc.reshape(-1, 1),
        W1T[:, :_D], W1T[:, _D:2 * _D], W1T[:, 2 * _D:], b1.reshape(-1, 1),
        W2.T, b2.reshape(-1, 1),
        W_out.T, b_out.reshape(1, 1),
    )
    return out3.reshape(_B, 1)


# speedup vs baseline: 21.6706x; 1.1246x over previous
import functools

import jax
import jax.numpy as jnp
from jax import lax
from jax.experimental import pallas as pl
from jax.experimental.pallas import tpu as pltpu
from jax.experimental.pallas import tpu_sc as plsc

_B = 16384
_D = 32
_V = 1000000
_PACK = 4
_PD = _D * _PACK
_NC = 2
_NS = 16
_NW = _NC * _NS
_BPW = _B // _NW

_PR = 262144
_CK = 4096
_CG = _PR // _CK


def _compact_body(u0, u1, u2, u3, i0, i1, i2, i3, uout_ref, iout_ref):
    f32 = jnp.float32
    eye = (lax.broadcasted_iota(jnp.int32, (_PD, _PD), 0) ==
           lax.broadcasted_iota(jnp.int32, (_PD, _PD), 1)).astype(f32)

    def place(refs, out_ref):
        x = jnp.concatenate([r[...] for r in refs], axis=0)
        out_ref[...] = lax.dot_general(x, eye, (((0,), (0,)), ((), ())),
                                       preferred_element_type=f32)

    place([u0, u1, u2, u3], uout_ref)
    place([i0, i1, i2, i3], iout_ref)


def _compact(utabT, itabT):
    in_specs = []
    last_blk = (_V - 1) // _CK
    for _ in range(2):
        for k in range(_PACK):
            in_specs.append(
                pl.BlockSpec(
                    (_D, _CK),
                    functools.partial(
                        lambda i, kk: (0, jnp.minimum(i + kk * _CG, last_blk)),
                        kk=k)))
    return pl.pallas_call(
        _compact_body,
        grid=(_CG,),
        in_specs=in_specs,
        out_specs=[
            pl.BlockSpec((_CK, _PD), lambda i: (i, 0)),
            pl.BlockSpec((_CK, _PD), lambda i: (i, 0)),
        ],
        out_shape=[
            jax.ShapeDtypeStruct((_PR, _PD), jnp.float32),
            jax.ShapeDtypeStruct((_PR, _PD), jnp.float32),
        ],
        compiler_params=pltpu.CompilerParams(fuse_transposed_lhs_in_matmul=True),
    )(utabT, utabT, utabT, utabT, itabT, itabT, itabT, itabT)


def _gather_body(user_tab, item_tab, uidx, iidx, uout, iout,
                 uidx_v, iidx_v, rows_v, sem):
    wid = lax.axis_index("s") * _NC + lax.axis_index("c")
    base = wid * _BPW
    pltpu.sync_copy(uidx.at[pl.ds(base, _BPW)], uidx_v)
    pltpu.sync_copy(iidx.at[pl.ds(base, _BPW)], iidx_v)
    for c in range(_BPW // 16):
        sl = pl.ds(c * 16, 16)
        uidx_v[sl] = jnp.bitwise_and(uidx_v[sl], _PR - 1)
        iidx_v[sl] = jnp.bitwise_and(iidx_v[sl], _PR - 1)
    pltpu.async_copy(user_tab.at[uidx_v], rows_v, sem).wait()
    pltpu.sync_copy(rows_v, uout.at[pl.ds(base, _BPW)])
    pltpu.async_copy(item_tab.at[iidx_v], rows_v, sem).wait()
    pltpu.sync_copy(rows_v, iout.at[pl.ds(base, _BPW)])


@functools.lru_cache(maxsize=None)
def _build_gather2():
    mesh = plsc.VectorSubcoreMesh(
        core_axis_name="c", subcore_axis_name="s",
        num_cores=_NC, num_subcores=_NS,
    )
    return pl.kernel(
        _gather_body,
        out_type=(
            jax.ShapeDtypeStruct((_B, _PD), jnp.float32),
            jax.ShapeDtypeStruct((_B, _PD), jnp.float32),
        ),
        mesh=mesh,
        scratch_types=[
            pltpu.VMEM((_BPW,), jnp.int32),
            pltpu.VMEM((_BPW,), jnp.int32),
            pltpu.VMEM((_BPW, _PD), jnp.float32),
            pltpu.SemaphoreType.DMA,
        ],
    )


_BS = 2048
_NB = _B // _BS


def _mlp_body(descT_ref, uraw_ref, iraw_ref, uoff_ref, ioff_ref,
              wdT_ref, bdT_ref, w1uT_ref, w1iT_ref, w1dT_ref, b1T_ref,
              w2T_ref, b2T_ref, woT_ref, bo_ref, out_ref):
    f32 = jnp.float32
    uraw = uraw_ref[...]
    iraw = iraw_ref[...]
    uoff = lax.shift_right_logical(jnp.transpose(uoff_ref[...]), 18)
    ioff = lax.shift_right_logical(jnp.transpose(ioff_ref[...]), 18)
    u = jnp.zeros((_BS, _D), f32)
    it = jnp.zeros((_BS, _D), f32)
    for k in range(_PACK):
        umask = (uoff == k).astype(f32)
        imask = (ioff == k).astype(f32)
        u = u + umask * uraw[:, k * _D:(k + 1) * _D]
        it = it + imask * iraw[:, k * _D:(k + 1) * _D]
    dT = lax.dot_general(wdT_ref[...], descT_ref[...], (((1,), (0,)), ((), ())),
                         preferred_element_type=f32)
    dT = jnp.maximum(dT + bdT_ref[...], 0.0)
    h1T = lax.dot_general(w1uT_ref[...], u, (((1,), (1,)), ((), ())),
                          preferred_element_type=f32)
    h1T = h1T + lax.dot_general(w1iT_ref[...], it, (((1,), (1,)), ((), ())),
                                preferred_element_type=f32)
    h1T = h1T + lax.dot_general(w1dT_ref[...], dT, (((1,), (0,)), ((), ())),
                                preferred_element_type=f32)
    h1T = jnp.maximum(h1T + b1T_ref[...], 0.0)
    h2T = lax.dot_general(w2T_ref[...], h1T, (((1,), (0,)), ((), ())),
                          preferred_element_type=f32)
    h2T = jnp.maximum(h2T + b2T_ref[...], 0.0)
    outT = lax.dot_general(woT_ref[...], h2T, (((1,), (0,)), ((), ())),
                           preferred_element_type=f32)
    out_ref[...] = (outT + bo_ref[...]).reshape(1, 1, _BS)


def _mlp(descT, u_raw, i_raw, uoff, ioff, wdT, bdT, w1uT, w1iT, w1dT, b1T,
         w2T, b2T, woT, bo):
    full = lambda shape: pl.BlockSpec(shape, lambda i: tuple(0 for _ in shape))
    return pl.pallas_call(
        _mlp_body,
        grid=(_NB,),
        in_specs=[
            pl.BlockSpec((300, _BS), lambda i: (0, i)),
            pl.BlockSpec((_BS, _PD), lambda i: (i, 0)),
            pl.BlockSpec((_BS, _PD), lambda i: (i, 0)),
            pl.BlockSpec((1, _BS), lambda i: (0, i)),
            pl.BlockSpec((1, _BS), lambda i: (0, i)),
            full((_D, 300)),
            full((_D, 1)),
            full((64, _D)),
            full((64, _D)),
            full((64, _D)),
            full((64, 1)),
            full((_D, 64)),
            full((_D, 1)),
            full((1, _D)),
            full((1, 1)),
        ],
        out_specs=pl.BlockSpec((1, 1, _BS), lambda i: (i, 0, 0)),
        out_shape=jax.ShapeDtypeStruct((_NB, 1, _BS), jnp.float32),
    )(descT, u_raw, i_raw, uoff, ioff, wdT, bdT, w1uT, w1iT, w1dT, b1T,
      w2T, b2T, woT, bo)


def kernel(user_input, item_input, description_input, user_table, item_table,
           W_desc, b_desc, W1, b1, W2, b2, W_out, b_out):
    utab4, itab4 = _compact(user_table.T, item_table.T)
    uidx = user_input.reshape(-1)
    iidx = item_input.reshape(-1)
    u_raw, i_raw = _build_gather2()(utab4, itab4, uidx, iidx)
    uoff = user_input.T
    ioff = item_input.T
    W1T = W1.T
    out3 = _mlp(
        description_input.T, u_raw, i_raw, uoff, ioff,
        W_desc.T, b_desc.reshape(-1, 1),
        W1T[:, :_D], W1T[:, _D:2 * _D], W1T[:, 2 * _D:], b1.reshape(-1, 1),
        W2.T, b2.reshape(-1, 1),
        W_out.T, b_out.reshape(1, 1),
    )
    return out3.reshape(_B, 1)
